# baseline (speedup 1.0000x reference)
import functools

import jax
import jax.numpy as jnp
from jax import lax
from jax.experimental import pallas as pl
from jax.experimental.pallas import tpu as pltpu
from jax.experimental.pallas import tpu_sc as plsc

_D = 1024
_NC = 2
_NS = 16
_NW = _NC * _NS
_K = 32


def _make_gather(n_tokens: int):
    bpw = n_tokens // _NW
    nchunk = bpw // _K

    @functools.partial(
        pl.kernel,
        out_type=jax.ShapeDtypeStruct((n_tokens, _D), jnp.float32),
        mesh=plsc.VectorSubcoreMesh(core_axis_name="c", subcore_axis_name="s"),
        scratch_types=[
            pltpu.VMEM((nchunk, _K), jnp.int32),
            pltpu.VMEM((_K, _D), jnp.float32),
            pltpu.VMEM((_K, _D), jnp.float32),
            pltpu.VMEM((_K, _D), jnp.float32),
            pltpu.SemaphoreType.DMA,
            pltpu.SemaphoreType.DMA,
            pltpu.SemaphoreType.DMA,
            pltpu.SemaphoreType.DMA,
            pltpu.SemaphoreType.DMA,
            pltpu.SemaphoreType.DMA,
        ],
    )
    def gather_kernel(idx_hbm, table_hbm, out_hbm, idx_v, rows0, rows1, rows2,
                      g0, g1, g2, o0, o1, o2):
        wid = lax.axis_index("s") * _NC + lax.axis_index("c")
        pltpu.sync_copy(idx_hbm.at[wid], idx_v)
        base = wid * bpw
        rows = (rows0, rows1, rows2)
        gs = (g0, g1, g2)
        os_ = (o0, o1, o2)
        nbuf = 3
        nsteps = (nchunk + nbuf - 1) // nbuf

        for b in range(nbuf):
            pltpu.async_copy(table_hbm.at[idx_v.at[b]], rows[b], gs[b])

        def step(i, carry):
            for b in range(nbuf):
                j = i * nbuf + b

                @pl.when(j < nchunk)
                def _():
                    pltpu.make_async_copy(
                        table_hbm.at[idx_v.at[0]], rows[b], gs[b]).wait()
                    pltpu.async_copy(
                        rows[b], out_hbm.at[pl.ds(base + j * _K, _K)], os_[b])
            for b in range(nbuf):
                j2 = i * nbuf + b + nbuf

                @pl.when(j2 < nchunk)
                def _():
                    pltpu.make_async_copy(
                        rows[b], out_hbm.at[pl.ds(base, _K)], os_[b]).wait()
                    pltpu.async_copy(table_hbm.at[idx_v.at[j2]], rows[b], gs[b])
            return carry

        lax.fori_loop(0, nsteps, step, 0)
        for b in range(nbuf):
            pltpu.make_async_copy(
                rows[b], out_hbm.at[pl.ds(base, _K)], os_[b]).wait()

    return gather_kernel


def kernel(x, table):
    batch, seq = x.shape
    n = batch * seq
    idx = x.reshape(_NW, n // (_NW * _K), _K).astype(jnp.int32)
    out = _make_gather(n)(idx, table)
    return out.reshape(batch, seq, _D)


# speedup vs baseline: 1.0735x; 1.0735x over previous
import functools

import jax
import jax.numpy as jnp
from jax import lax
from jax.experimental import pallas as pl
from jax.experimental.pallas import tpu as pltpu
from jax.experimental.pallas import tpu_sc as plsc

_D = 1024
_NC = 2
_NS = 16
_NW = _NC * _NS
_K = 16
_NBUF = 4


def _make_gather(n_tokens: int):
    bpw = n_tokens // _NW
    nchunk = bpw // _K

    @functools.partial(
        pl.kernel,
        out_type=jax.ShapeDtypeStruct((n_tokens, _D), jnp.float32),
        mesh=plsc.VectorSubcoreMesh(core_axis_name="c", subcore_axis_name="s"),
        scratch_types=(
            [pltpu.VMEM((nchunk, _K), jnp.int32)]
            + [pltpu.VMEM((_K, _D), jnp.float32)] * _NBUF
            + [pltpu.SemaphoreType.DMA] * (2 * _NBUF)
        ),
    )
    def gather_kernel(idx_hbm, table_hbm, out_hbm, idx_v, *bufs):
        rows = bufs[:_NBUF]
        gs = bufs[_NBUF:2 * _NBUF]
        os_ = bufs[2 * _NBUF:]
        wid = lax.axis_index("s") * _NC + lax.axis_index("c")
        pltpu.sync_copy(idx_hbm.at[wid], idx_v)
        base = wid * bpw

        for b in range(_NBUF):
            pltpu.async_copy(table_hbm.at[idx_v.at[b]], rows[b], gs[b])

        def step(i, carry):
            for b in range(_NBUF):
                j = i * _NBUF + b
                j2 = j + _NBUF
                pltpu.make_async_copy(
                    table_hbm.at[idx_v.at[0]], rows[b], gs[b]).wait()
                pltpu.async_copy(
                    rows[b], out_hbm.at[pl.ds(base + j * _K, _K)], os_[b])

                @pl.when(j2 < nchunk)
                def _():
                    pltpu.make_async_copy(
                        rows[b], out_hbm.at[pl.ds(base, _K)], os_[b]).wait()
                    pltpu.async_copy(table_hbm.at[idx_v.at[j2]], rows[b], gs[b])
            return carry

        lax.fori_loop(0, nchunk // _NBUF, step, 0)
        for b in range(_NBUF):
            pltpu.make_async_copy(
                rows[b], out_hbm.at[pl.ds(base, _K)], os_[b]).wait()

    return gather_kernel


def kernel(x, table):
    batch, seq = x.shape
    n = batch * seq
    idx = x.reshape(_NW, n // (_NW * _K), _K).astype(jnp.int32)
    out = _make_gather(n)(idx, table)
    return out.reshape(batch, seq, _D)


# speedup vs baseline: 1.0793x; 1.0053x over previous
import functools

import jax
import jax.numpy as jnp
from jax import lax
from jax.experimental import pallas as pl
from jax.experimental.pallas import tpu as pltpu
from jax.experimental.pallas import tpu_sc as plsc

_D = 1024
_NC = 2
_NS = 16
_NW = _NC * _NS
_K = 8
_NBUF = 8


def _make_gather(n_tokens: int):
    bpw = n_tokens // _NW
    nchunk = bpw // _K

    @functools.partial(
        pl.kernel,
        out_type=jax.ShapeDtypeStruct((n_tokens, _D), jnp.float32),
        mesh=plsc.VectorSubcoreMesh(core_axis_name="c", subcore_axis_name="s"),
        scratch_types=(
            [pltpu.VMEM((nchunk, _K), jnp.int32)]
            + [pltpu.VMEM((_K, _D), jnp.float32)] * _NBUF
            + [pltpu.SemaphoreType.DMA] * (2 * _NBUF)
        ),
    )
    def gather_kernel(idx_hbm, table_hbm, out_hbm, idx_v, *bufs):
        rows = bufs[:_NBUF]
        gs = bufs[_NBUF:2 * _NBUF]
        os_ = bufs[2 * _NBUF:]
        wid = lax.axis_index("s") * _NC + lax.axis_index("c")
        pltpu.sync_copy(idx_hbm.at[wid], idx_v)
        base = wid * bpw

        for b in range(_NBUF):
            pltpu.async_copy(table_hbm.at[idx_v.at[b]], rows[b], gs[b])

        def step(i, carry):
            for b in range(_NBUF):
                j = i * _NBUF + b
                j2 = j + _NBUF
                pltpu.make_async_copy(
                    table_hbm.at[idx_v.at[0]], rows[b], gs[b]).wait()
                pltpu.async_copy(
                    rows[b], out_hbm.at[pl.ds(base + j * _K, _K)], os_[b])

                @pl.when(j2 < nchunk)
                def _():
                    pltpu.make_async_copy(
                        rows[b], out_hbm.at[pl.ds(base, _K)], os_[b]).wait()
                    pltpu.async_copy(table_hbm.at[idx_v.at[j2]], rows[b], gs[b])
            return carry

        lax.fori_loop(0, nchunk // _NBUF, step, 0)
        for b in range(_NBUF):
            pltpu.make_async_copy(
                rows[b], out_hbm.at[pl.ds(base, _K)], os_[b]).wait()

    return gather_kernel


def kernel(x, table):
    batch, seq = x.shape
    n = batch * seq
    idx = x.reshape(_NW, n // (_NW * _K), _K).astype(jnp.int32)
    out = _make_gather(n)(idx, table)
    return out.reshape(batch, seq, _D)
